# Initial kernel scaffold; baseline (speedup 1.0000x reference)
import jax
import jax.numpy as jnp
from jax.experimental import pallas as pl


def kernel(X, y, W_rf, b_rf, pca_mean, pca_comp, W0, b0, W1, b1, W2, b2, W3, b3, nn_bias0, nn_bias1):
    raise NotImplementedError("write your pallas kernel here")


# speedup vs baseline: 10.2764x; 10.2764x over previous
import functools

import jax
import jax.numpy as jnp
from jax import lax
from jax.experimental import pallas as pl
from jax.experimental.pallas import tpu as pltpu
from jax.experimental.pallas import tpu_sc as plsc

N = 4096
D_IN = 128
D_RF = 512
D_PCA = 256
HID = 256
NCLS = 10
CLIP = 5.0
KBATCH = 128
NPAD = 16

RQ = 512
KB = 512
PREC = None

NW = 32
ROWS_W = N // NW
L = 16


def _mm(a, b, dims):
    return lax.dot_general(a, b, (dims, ((), ())), precision=PREC,
                           preferred_element_type=jnp.float32)


def _mlp_body(x_ref, wrf_ref, brf_ref, pmean_ref, pcomp_ref,
              w0_ref, b0_ref, w1_ref, b1_ref, w2_ref, b2_ref,
              w3_ref, b3_ref, h1_ref, logit_ref):
    x = x_ref[...]
    h = _mm(x, wrf_ref[...], ((1,), (1,))) + brf_ref[...]
    h = h - pmean_ref[...]
    h = _mm(h, pcomp_ref[...], ((1,), (1,)))
    h = jnp.clip(h, -CLIP, CLIP)
    res = h
    h = jnp.maximum(_mm(h, w0_ref[...], ((1,), (0,))) + b0_ref[...], 0.0)
    h = _mm(h, w1_ref[...], ((1,), (0,))) + b1_ref[...]
    h = jnp.maximum(h + res, 0.0)
    h = jnp.maximum(_mm(h, w2_ref[...], ((1,), (0,))) + b2_ref[...], 0.0)
    h1_ref[...] = h
    logit_ref[...] = _mm(w3_ref[...], h, ((0,), (1,))) + b3_ref[...]


def _argmin_stream(q_ref, k_ref, out_ref):
    q = q_ref[...]
    q2 = jnp.sum(q * q, axis=1, keepdims=True)
    row = lax.broadcasted_iota(jnp.int32, (RQ, KB), 0)
    col = lax.broadcasted_iota(jnp.int32, (RQ, KB), 1)
    gmin = jnp.full((RQ, 1), jnp.inf, dtype=jnp.float32)
    gidx = jnp.zeros((RQ, 1), dtype=jnp.int32)
    for kb in range(N // KB):
        k = k_ref[kb * KB:(kb + 1) * KB, :]
        k2 = jnp.sum(k * k, axis=1)[None, :]
        g = _mm(q, k, ((1,), (1,)))
        d2 = q2 + k2 - 2.0 * g
        if kb == 0:
            d2 = jnp.where(col == row % KBATCH, jnp.inf, d2)
        bmin = jnp.min(d2, axis=1, keepdims=True)
        bidx = jnp.min(jnp.where(d2 == bmin, col + kb * KB, N),
                       axis=1, keepdims=True)
        upd = bmin < gmin
        gidx = jnp.where(upd, bidx, gidx)
        gmin = jnp.where(upd, bmin, gmin)
    out_ref[0, 0, :] = gidx[:, 0]


def _knn_body(q0_ref, q1_ref, k0_ref, k1_ref, i0_ref, i1_ref):
    _argmin_stream(q0_ref, k0_ref, i0_ref)
    _argmin_stream(q1_ref, k1_ref, i1_ref)


@functools.cache
def _sc_bias_kernel():
    @functools.partial(
        pl.kernel,
        mesh=plsc.VectorSubcoreMesh(core_axis_name="c", subcore_axis_name="s"),
        out_type=jax.ShapeDtypeStruct((NPAD, N), jnp.float32),
        scratch_types=[
            pltpu.VMEM((NPAD, ROWS_W), jnp.float32),
            pltpu.VMEM((ROWS_W,), jnp.int32),
            pltpu.VMEM((ROWS_W,), jnp.int32),
            pltpu.VMEM((ROWS_W,), jnp.int32),
            pltpu.VMEM((ROWS_W,), jnp.int32),
            pltpu.VMEM((L,), jnp.float32),
            pltpu.VMEM((L,), jnp.float32),
            pltpu.SemaphoreType.DMA,
        ],
    )
    def _sc_bias(logits_hbm, idx0_hbm, idx1_hbm, y_hbm, b0_hbm, b1_hbm,
                 out_hbm, lg_v, i0_v, i1_v, lab0_v, lab1_v, b0_v, b1_v, sem):
        wid = lax.axis_index("s") * 2 + lax.axis_index("c")
        base = wid * ROWS_W
        pltpu.sync_copy(logits_hbm.at[:, pl.ds(base, ROWS_W)], lg_v)
        pltpu.sync_copy(idx0_hbm.at[pl.ds(base, ROWS_W)], i0_v)
        pltpu.sync_copy(idx1_hbm.at[pl.ds(base, ROWS_W)], i1_v)
        pltpu.async_copy(y_hbm.at[i0_v], lab0_v, sem).wait()
        pltpu.async_copy(y_hbm.at[i1_v], lab1_v, sem).wait()
        pltpu.sync_copy(b0_hbm, b0_v)
        pltpu.sync_copy(b1_hbm, b1_v)
        b0 = b0_v[...]
        b1 = b1_v[...]
        zero = jnp.zeros((L,), jnp.float32)
        for i in range(ROWS_W // L):
            sl = pl.ds(i * L, L)
            l0 = lab0_v[sl]
            l1 = lab1_v[sl]
            for c in range(NCLS):
                add = (jnp.where(l0 == c, b0, zero)
                       + jnp.where(l1 == c, b1, zero))
                lg_v[c, sl] = lg_v[c, sl] + add
        pltpu.sync_copy(lg_v, out_hbm.at[:, pl.ds(base, ROWS_W)])

    return _sc_bias


def _row2d(v):
    return v.reshape(1, -1)


def kernel(X, y, W_rf, b_rf, pca_mean, pca_comp, W0, b0, W1, b1,
           W2, b2, W3, b3, nn_bias0, nn_bias1):
    nq = N // RQ
    w3p = jnp.zeros((HID, NPAD), jnp.float32).at[:, :NCLS].set(W3)
    b3p = jnp.zeros((NPAD, 1), jnp.float32).at[:NCLS, 0].set(b3)

    full = lambda shape: pl.BlockSpec(shape, lambda i: (0, 0))
    h1, logits = pl.pallas_call(
        _mlp_body,
        grid=(nq,),
        in_specs=[
            pl.BlockSpec((RQ, D_IN), lambda i: (i, 0)),
            full((D_RF, D_IN)), full((1, D_RF)), full((1, D_RF)),
            full((D_PCA, D_RF)),
            full((HID, HID)), full((1, HID)),
            full((HID, HID)), full((1, HID)),
            full((HID, HID)), full((1, HID)),
            full((HID, NPAD)), full((NPAD, 1)),
        ],
        out_specs=[
            pl.BlockSpec((RQ, HID), lambda i: (i, 0)),
            pl.BlockSpec((NPAD, RQ), lambda i: (0, i)),
        ],
        out_shape=[
            jax.ShapeDtypeStruct((N, HID), jnp.float32),
            jax.ShapeDtypeStruct((NPAD, N), jnp.float32),
        ],
        compiler_params=pltpu.CompilerParams(
            dimension_semantics=("arbitrary",)),
    )(X, W_rf, _row2d(b_rf), _row2d(pca_mean), pca_comp,
      W0, _row2d(b0), W1, _row2d(b1), W2, _row2d(b2), w3p, b3p)

    idx0, idx1 = pl.pallas_call(
        _knn_body,
        grid=(nq,),
        in_specs=[
            pl.BlockSpec((RQ, D_IN), lambda i: (i, 0)),
            pl.BlockSpec((RQ, HID), lambda i: (i, 0)),
            full((N, D_IN)), full((N, HID)),
        ],
        out_specs=[
            pl.BlockSpec((1, 1, RQ), lambda i: (i, 0, 0)),
            pl.BlockSpec((1, 1, RQ), lambda i: (i, 0, 0)),
        ],
        out_shape=[
            jax.ShapeDtypeStruct((nq, 1, RQ), jnp.int32),
            jax.ShapeDtypeStruct((nq, 1, RQ), jnp.int32),
        ],
        compiler_params=pltpu.CompilerParams(
            dimension_semantics=("arbitrary",)),
    )(X, h1, X, h1)

    b0v = jnp.full((L,), nn_bias0, dtype=jnp.float32)
    b1v = jnp.full((L,), nn_bias1, dtype=jnp.float32)
    out16 = _sc_bias_kernel()(logits, idx0.reshape(N), idx1.reshape(N),
                              y, b0v, b1v)
    return out16[:NCLS, :].T


# speedup vs baseline: 11.8814x; 1.1562x over previous
import functools

import jax
import jax.numpy as jnp
from jax import lax
from jax.experimental import pallas as pl
from jax.experimental.pallas import tpu as pltpu
from jax.experimental.pallas import tpu_sc as plsc

N = 4096
D_IN = 128
D_RF = 512
D_PCA = 256
HID = 256
NCLS = 10
CLIP = 5.0
KBATCH = 128
NPAD = 16

RQ = 512
KB = 512
PREC = None

NW = 32
ROWS_W = N // NW
L = 16


def _mm(a, b, dims):
    return lax.dot_general(a, b, (dims, ((), ())), precision=PREC,
                           preferred_element_type=jnp.float32)


def _mlp_body(x_ref, wrf_ref, brf_ref, pmean_ref, pcomp_ref,
              w0_ref, b0_ref, w1_ref, b1_ref, w2_ref, b2_ref,
              w3_ref, b3_ref, h1_ref, logit_ref):
    x = x_ref[...]
    h = _mm(x, wrf_ref[...], ((1,), (1,))) + brf_ref[...]
    h = h - pmean_ref[...]
    h = _mm(h, pcomp_ref[...], ((1,), (1,)))
    h = jnp.clip(h, -CLIP, CLIP)
    res = h
    h = jnp.maximum(_mm(h, w0_ref[...], ((1,), (0,))) + b0_ref[...], 0.0)
    h = _mm(h, w1_ref[...], ((1,), (0,))) + b1_ref[...]
    h = jnp.maximum(h + res, 0.0)
    h = jnp.maximum(_mm(h, w2_ref[...], ((1,), (0,))) + b2_ref[...], 0.0)
    h1_ref[...] = h
    logit_ref[...] = _mm(w3_ref[...], h, ((0,), (1,))) + b3_ref[...]


def _argmin_stream(q_ref, k_ref, out_ref):
    q = q_ref[...]
    qm2 = q * -2.0
    q2 = jnp.sum(q * q, axis=1, keepdims=True)
    row = lax.broadcasted_iota(jnp.int32, (RQ, KB), 0)
    col = lax.broadcasted_iota(jnp.int32, (RQ, KB), 1)
    colf = col.astype(jnp.float32)
    gmin = jnp.full((RQ, 1), jnp.inf, dtype=jnp.float32)
    gidx = jnp.zeros((RQ, 1), dtype=jnp.float32)
    for kb in range(N // KB):
        k = k_ref[kb * KB:(kb + 1) * KB, :]
        k2 = jnp.sum(k * k, axis=1)[None, :]
        g = _mm(qm2, k, ((1,), (1,)))
        d2 = q2 + k2 + g
        if kb == 0:
            d2 = jnp.where(col == row % KBATCH, jnp.inf, d2)
        bmin = jnp.min(d2, axis=1, keepdims=True)
        bidx = jnp.min(jnp.where(d2 == bmin, colf, float(N)),
                       axis=1, keepdims=True) + float(kb * KB)
        upd = bmin < gmin
        gidx = jnp.where(upd, bidx, gidx)
        gmin = jnp.where(upd, bmin, gmin)
    out_ref[0, 0, :] = gidx[:, 0].astype(jnp.int32)


def _knn_body(q0_ref, q1_ref, k0_ref, k1_ref, i0_ref, i1_ref):
    _argmin_stream(q0_ref, k0_ref, i0_ref)
    _argmin_stream(q1_ref, k1_ref, i1_ref)


@functools.cache
def _sc_bias_kernel():
    @functools.partial(
        pl.kernel,
        mesh=plsc.VectorSubcoreMesh(core_axis_name="c", subcore_axis_name="s"),
        out_type=jax.ShapeDtypeStruct((NPAD, N), jnp.float32),
        scratch_types=[
            pltpu.VMEM((NPAD, ROWS_W), jnp.float32),
            pltpu.VMEM((ROWS_W,), jnp.int32),
            pltpu.VMEM((ROWS_W,), jnp.int32),
            pltpu.VMEM((ROWS_W,), jnp.int32),
            pltpu.VMEM((ROWS_W,), jnp.int32),
            pltpu.VMEM((L,), jnp.float32),
            pltpu.VMEM((L,), jnp.float32),
            pltpu.SemaphoreType.DMA,
        ],
    )
    def _sc_bias(logits_hbm, idx0_hbm, idx1_hbm, y_hbm, b0_hbm, b1_hbm,
                 out_hbm, lg_v, i0_v, i1_v, lab0_v, lab1_v, b0_v, b1_v, sem):
        wid = lax.axis_index("s") * 2 + lax.axis_index("c")
        base = wid * ROWS_W
        pltpu.sync_copy(logits_hbm.at[:, pl.ds(base, ROWS_W)], lg_v)
        pltpu.sync_copy(idx0_hbm.at[pl.ds(base, ROWS_W)], i0_v)
        pltpu.sync_copy(idx1_hbm.at[pl.ds(base, ROWS_W)], i1_v)
        pltpu.async_copy(y_hbm.at[i0_v], lab0_v, sem).wait()
        pltpu.async_copy(y_hbm.at[i1_v], lab1_v, sem).wait()
        pltpu.sync_copy(b0_hbm, b0_v)
        pltpu.sync_copy(b1_hbm, b1_v)
        b0 = b0_v[...]
        b1 = b1_v[...]
        zero = jnp.zeros((L,), jnp.float32)
        for i in range(ROWS_W // L):
            sl = pl.ds(i * L, L)
            l0 = lab0_v[sl]
            l1 = lab1_v[sl]
            for c in range(NCLS):
                add = (jnp.where(l0 == c, b0, zero)
                       + jnp.where(l1 == c, b1, zero))
                lg_v[c, sl] = lg_v[c, sl] + add
        pltpu.sync_copy(lg_v, out_hbm.at[:, pl.ds(base, ROWS_W)])

    return _sc_bias


def _row2d(v):
    return v.reshape(1, -1)


def kernel(X, y, W_rf, b_rf, pca_mean, pca_comp, W0, b0, W1, b1,
           W2, b2, W3, b3, nn_bias0, nn_bias1):
    nq = N // RQ
    w3p = jnp.zeros((HID, NPAD), jnp.float32).at[:, :NCLS].set(W3)
    b3p = jnp.zeros((NPAD, 1), jnp.float32).at[:NCLS, 0].set(b3)

    full = lambda shape: pl.BlockSpec(shape, lambda i: (0, 0))
    h1, logits = pl.pallas_call(
        _mlp_body,
        grid=(nq,),
        in_specs=[
            pl.BlockSpec((RQ, D_IN), lambda i: (i, 0)),
            full((D_RF, D_IN)), full((1, D_RF)), full((1, D_RF)),
            full((D_PCA, D_RF)),
            full((HID, HID)), full((1, HID)),
            full((HID, HID)), full((1, HID)),
            full((HID, HID)), full((1, HID)),
            full((HID, NPAD)), full((NPAD, 1)),
        ],
        out_specs=[
            pl.BlockSpec((RQ, HID), lambda i: (i, 0)),
            pl.BlockSpec((NPAD, RQ), lambda i: (0, i)),
        ],
        out_shape=[
            jax.ShapeDtypeStruct((N, HID), jnp.float32),
            jax.ShapeDtypeStruct((NPAD, N), jnp.float32),
        ],
        compiler_params=pltpu.CompilerParams(
            dimension_semantics=("arbitrary",)),
    )(X, W_rf, _row2d(b_rf), _row2d(pca_mean), pca_comp,
      W0, _row2d(b0), W1, _row2d(b1), W2, _row2d(b2), w3p, b3p)

    idx0, idx1 = pl.pallas_call(
        _knn_body,
        grid=(nq,),
        in_specs=[
            pl.BlockSpec((RQ, D_IN), lambda i: (i, 0)),
            pl.BlockSpec((RQ, HID), lambda i: (i, 0)),
            full((N, D_IN)), full((N, HID)),
        ],
        out_specs=[
            pl.BlockSpec((1, 1, RQ), lambda i: (i, 0, 0)),
            pl.BlockSpec((1, 1, RQ), lambda i: (i, 0, 0)),
        ],
        out_shape=[
            jax.ShapeDtypeStruct((nq, 1, RQ), jnp.int32),
            jax.ShapeDtypeStruct((nq, 1, RQ), jnp.int32),
        ],
        compiler_params=pltpu.CompilerParams(
            dimension_semantics=("arbitrary",)),
    )(X, h1, X, h1)

    b0v = jnp.full((L,), nn_bias0, dtype=jnp.float32)
    b1v = jnp.full((L,), nn_bias1, dtype=jnp.float32)
    out16 = _sc_bias_kernel()(logits, idx0.reshape(N), idx1.reshape(N),
                              y, b0v, b1v)
    return out16[:NCLS, :].T


# speedup vs baseline: 12.8106x; 1.0782x over previous
import functools

import jax
import jax.numpy as jnp
from jax import lax
from jax.experimental import pallas as pl
from jax.experimental.pallas import tpu as pltpu
from jax.experimental.pallas import tpu_sc as plsc

N = 4096
D_IN = 128
D_RF = 512
D_PCA = 256
HID = 256
NCLS = 10
CLIP = 5.0
KBATCH = 128
NPAD = 16

RQ = 1024
KB = 512
PREC = None

NW = 32
ROWS_W = N // NW
L = 16


def _mm(a, b, dims):
    return lax.dot_general(a, b, (dims, ((), ())), precision=PREC,
                           preferred_element_type=jnp.float32)


def _mlp_body(x_ref, wrf_ref, brf_ref, pmean_ref, pcomp_ref,
              w0_ref, b0_ref, w1_ref, b1_ref, w2_ref, b2_ref,
              w3_ref, b3_ref, h1_ref, logit_ref):
    x = x_ref[...]
    h = _mm(x, wrf_ref[...], ((1,), (1,))) + brf_ref[...]
    h = h - pmean_ref[...]
    h = _mm(h, pcomp_ref[...], ((1,), (1,)))
    h = jnp.clip(h, -CLIP, CLIP)
    res = h
    h = jnp.maximum(_mm(h, w0_ref[...], ((1,), (0,))) + b0_ref[...], 0.0)
    h = _mm(h, w1_ref[...], ((1,), (0,))) + b1_ref[...]
    h = jnp.maximum(h + res, 0.0)
    h = jnp.maximum(_mm(h, w2_ref[...], ((1,), (0,))) + b2_ref[...], 0.0)
    h1_ref[...] = h
    logit_ref[...] = _mm(w3_ref[...], h, ((0,), (1,))) + b3_ref[...]


def _argmin_stream(q_ref, k_ref, out_ref):
    q = q_ref[...]
    qm2 = q * -2.0
    q2 = jnp.sum(q * q, axis=1, keepdims=True)
    row = lax.broadcasted_iota(jnp.int32, (RQ, KB), 0)
    col = lax.broadcasted_iota(jnp.int32, (RQ, KB), 1)
    colf = col.astype(jnp.float32)
    gmin = jnp.full((RQ, 1), jnp.inf, dtype=jnp.float32)
    gidx = jnp.zeros((RQ, 1), dtype=jnp.float32)
    for kb in range(N // KB):
        k = k_ref[kb * KB:(kb + 1) * KB, :]
        k2 = jnp.sum(k * k, axis=1)[None, :]
        g = _mm(qm2, k, ((1,), (1,)))
        d2 = q2 + k2 + g
        if kb == 0:
            d2 = jnp.where(col == row % KBATCH, jnp.inf, d2)
        bmin = jnp.min(d2, axis=1, keepdims=True)
        bidx = jnp.min(jnp.where(d2 == bmin, colf, float(N)),
                       axis=1, keepdims=True) + float(kb * KB)
        upd = bmin < gmin
        gidx = jnp.where(upd, bidx, gidx)
        gmin = jnp.where(upd, bmin, gmin)
    out_ref[0, 0, :] = gidx[:, 0].astype(jnp.int32)


def _knn_body(q0_ref, q1_ref, k0_ref, k1_ref, i0_ref, i1_ref):
    _argmin_stream(q0_ref, k0_ref, i0_ref)
    _argmin_stream(q1_ref, k1_ref, i1_ref)


@functools.cache
def _sc_bias_kernel():
    @functools.partial(
        pl.kernel,
        mesh=plsc.VectorSubcoreMesh(core_axis_name="c", subcore_axis_name="s"),
        out_type=jax.ShapeDtypeStruct((NPAD, N), jnp.float32),
        scratch_types=[
            pltpu.VMEM((NPAD, ROWS_W), jnp.float32),
            pltpu.VMEM((ROWS_W,), jnp.int32),
            pltpu.VMEM((ROWS_W,), jnp.int32),
            pltpu.VMEM((ROWS_W,), jnp.int32),
            pltpu.VMEM((ROWS_W,), jnp.int32),
            pltpu.VMEM((L,), jnp.float32),
            pltpu.VMEM((L,), jnp.float32),
            pltpu.SemaphoreType.DMA,
        ],
    )
    def _sc_bias(logits_hbm, idx0_hbm, idx1_hbm, y_hbm, b0_hbm, b1_hbm,
                 out_hbm, lg_v, i0_v, i1_v, lab0_v, lab1_v, b0_v, b1_v, sem):
        wid = lax.axis_index("s") * 2 + lax.axis_index("c")
        base = wid * ROWS_W
        pltpu.sync_copy(logits_hbm.at[:, pl.ds(base, ROWS_W)], lg_v)
        pltpu.sync_copy(idx0_hbm.at[pl.ds(base, ROWS_W)], i0_v)
        pltpu.sync_copy(idx1_hbm.at[pl.ds(base, ROWS_W)], i1_v)
        pltpu.async_copy(y_hbm.at[i0_v], lab0_v, sem).wait()
        pltpu.async_copy(y_hbm.at[i1_v], lab1_v, sem).wait()
        pltpu.sync_copy(b0_hbm, b0_v)
        pltpu.sync_copy(b1_hbm, b1_v)
        b0 = b0_v[...]
        b1 = b1_v[...]
        zero = jnp.zeros((L,), jnp.float32)
        for i in range(ROWS_W // L):
            sl = pl.ds(i * L, L)
            l0 = lab0_v[sl]
            l1 = lab1_v[sl]
            for c in range(NCLS):
                add = (jnp.where(l0 == c, b0, zero)
                       + jnp.where(l1 == c, b1, zero))
                lg_v[c, sl] = lg_v[c, sl] + add
        pltpu.sync_copy(lg_v, out_hbm.at[:, pl.ds(base, ROWS_W)])

    return _sc_bias


def _row2d(v):
    return v.reshape(1, -1)


def kernel(X, y, W_rf, b_rf, pca_mean, pca_comp, W0, b0, W1, b1,
           W2, b2, W3, b3, nn_bias0, nn_bias1):
    nq = N // RQ
    w3p = jnp.zeros((HID, NPAD), jnp.float32).at[:, :NCLS].set(W3)
    b3p = jnp.zeros((NPAD, 1), jnp.float32).at[:NCLS, 0].set(b3)

    full = lambda shape: pl.BlockSpec(shape, lambda i: (0, 0))
    h1, logits = pl.pallas_call(
        _mlp_body,
        grid=(nq,),
        in_specs=[
            pl.BlockSpec((RQ, D_IN), lambda i: (i, 0)),
            full((D_RF, D_IN)), full((1, D_RF)), full((1, D_RF)),
            full((D_PCA, D_RF)),
            full((HID, HID)), full((1, HID)),
            full((HID, HID)), full((1, HID)),
            full((HID, HID)), full((1, HID)),
            full((HID, NPAD)), full((NPAD, 1)),
        ],
        out_specs=[
            pl.BlockSpec((RQ, HID), lambda i: (i, 0)),
            pl.BlockSpec((NPAD, RQ), lambda i: (0, i)),
        ],
        out_shape=[
            jax.ShapeDtypeStruct((N, HID), jnp.float32),
            jax.ShapeDtypeStruct((NPAD, N), jnp.float32),
        ],
        compiler_params=pltpu.CompilerParams(
            dimension_semantics=("arbitrary",)),
    )(X, W_rf, _row2d(b_rf), _row2d(pca_mean), pca_comp,
      W0, _row2d(b0), W1, _row2d(b1), W2, _row2d(b2), w3p, b3p)

    idx0, idx1 = pl.pallas_call(
        _knn_body,
        grid=(nq,),
        in_specs=[
            pl.BlockSpec((RQ, D_IN), lambda i: (i, 0)),
            pl.BlockSpec((RQ, HID), lambda i: (i, 0)),
            full((N, D_IN)), full((N, HID)),
        ],
        out_specs=[
            pl.BlockSpec((1, 1, RQ), lambda i: (i, 0, 0)),
            pl.BlockSpec((1, 1, RQ), lambda i: (i, 0, 0)),
        ],
        out_shape=[
            jax.ShapeDtypeStruct((nq, 1, RQ), jnp.int32),
            jax.ShapeDtypeStruct((nq, 1, RQ), jnp.int32),
        ],
        compiler_params=pltpu.CompilerParams(
            dimension_semantics=("arbitrary",)),
    )(X, h1, X, h1)

    b0v = jnp.full((L,), nn_bias0, dtype=jnp.float32)
    b1v = jnp.full((L,), nn_bias1, dtype=jnp.float32)
    out16 = _sc_bias_kernel()(logits, idx0.reshape(N), idx1.reshape(N),
                              y, b0v, b1v)
    return out16[:NCLS, :].T


# speedup vs baseline: 12.8422x; 1.0025x over previous
import functools

import jax
import jax.numpy as jnp
from jax import lax
from jax.experimental import pallas as pl
from jax.experimental.pallas import tpu as pltpu
from jax.experimental.pallas import tpu_sc as plsc

N = 4096
D_IN = 128
D_RF = 512
D_PCA = 256
HID = 256
NCLS = 10
CLIP = 5.0
KBATCH = 128
NPAD = 16

RQ = 2048
KB = 512
PREC = None

NW = 32
ROWS_W = N // NW
L = 16


def _mm(a, b, dims):
    return lax.dot_general(a, b, (dims, ((), ())), precision=PREC,
                           preferred_element_type=jnp.float32)


def _mlp_body(x_ref, wrf_ref, brf_ref, pmean_ref, pcomp_ref,
              w0_ref, b0_ref, w1_ref, b1_ref, w2_ref, b2_ref,
              w3_ref, b3_ref, h1_ref, logit_ref):
    x = x_ref[...]
    h = _mm(x, wrf_ref[...], ((1,), (1,))) + brf_ref[...]
    h = h - pmean_ref[...]
    h = _mm(h, pcomp_ref[...], ((1,), (1,)))
    h = jnp.clip(h, -CLIP, CLIP)
    res = h
    h = jnp.maximum(_mm(h, w0_ref[...], ((1,), (0,))) + b0_ref[...], 0.0)
    h = _mm(h, w1_ref[...], ((1,), (0,))) + b1_ref[...]
    h = jnp.maximum(h + res, 0.0)
    h = jnp.maximum(_mm(h, w2_ref[...], ((1,), (0,))) + b2_ref[...], 0.0)
    h1_ref[...] = h
    logit_ref[...] = _mm(w3_ref[...], h, ((0,), (1,))) + b3_ref[...]


def _argmin_stream(q_ref, k_ref, out_ref):
    q = q_ref[...]
    qm2 = q * -2.0
    q2 = jnp.sum(q * q, axis=1, keepdims=True)
    row = lax.broadcasted_iota(jnp.int32, (RQ, KB), 0)
    col = lax.broadcasted_iota(jnp.int32, (RQ, KB), 1)
    colf = col.astype(jnp.float32)
    gmin = jnp.full((RQ, 1), jnp.inf, dtype=jnp.float32)
    gidx = jnp.zeros((RQ, 1), dtype=jnp.float32)
    for kb in range(N // KB):
        k = k_ref[kb * KB:(kb + 1) * KB, :]
        k2 = jnp.sum(k * k, axis=1)[None, :]
        g = _mm(qm2, k, ((1,), (1,)))
        d2 = q2 + k2 + g
        if kb == 0:
            d2 = jnp.where(col == row % KBATCH, jnp.inf, d2)
        bmin = jnp.min(d2, axis=1, keepdims=True)
        bidx = jnp.min(jnp.where(d2 == bmin, colf, float(N)),
                       axis=1, keepdims=True) + float(kb * KB)
        upd = bmin < gmin
        gidx = jnp.where(upd, bidx, gidx)
        gmin = jnp.where(upd, bmin, gmin)
    out_ref[0, 0, :] = gidx[:, 0].astype(jnp.int32)


def _knn_body(q0_ref, q1_ref, k0_ref, k1_ref, i0_ref, i1_ref):
    _argmin_stream(q0_ref, k0_ref, i0_ref)
    _argmin_stream(q1_ref, k1_ref, i1_ref)


@functools.cache
def _sc_bias_kernel():
    @functools.partial(
        pl.kernel,
        mesh=plsc.VectorSubcoreMesh(core_axis_name="c", subcore_axis_name="s"),
        out_type=jax.ShapeDtypeStruct((NPAD, N), jnp.float32),
        scratch_types=[
            pltpu.VMEM((NPAD, ROWS_W), jnp.float32),
            pltpu.VMEM((ROWS_W,), jnp.int32),
            pltpu.VMEM((ROWS_W,), jnp.int32),
            pltpu.VMEM((ROWS_W,), jnp.int32),
            pltpu.VMEM((ROWS_W,), jnp.int32),
            pltpu.VMEM((L,), jnp.float32),
            pltpu.VMEM((L,), jnp.float32),
            pltpu.SemaphoreType.DMA,
        ],
    )
    def _sc_bias(logits_hbm, idx0_hbm, idx1_hbm, y_hbm, b0_hbm, b1_hbm,
                 out_hbm, lg_v, i0_v, i1_v, lab0_v, lab1_v, b0_v, b1_v, sem):
        wid = lax.axis_index("s") * 2 + lax.axis_index("c")
        base = wid * ROWS_W
        pltpu.sync_copy(logits_hbm.at[:, pl.ds(base, ROWS_W)], lg_v)
        pltpu.sync_copy(idx0_hbm.at[pl.ds(base, ROWS_W)], i0_v)
        pltpu.sync_copy(idx1_hbm.at[pl.ds(base, ROWS_W)], i1_v)
        pltpu.async_copy(y_hbm.at[i0_v], lab0_v, sem).wait()
        pltpu.async_copy(y_hbm.at[i1_v], lab1_v, sem).wait()
        pltpu.sync_copy(b0_hbm, b0_v)
        pltpu.sync_copy(b1_hbm, b1_v)
        b0 = b0_v[...]
        b1 = b1_v[...]
        zero = jnp.zeros((L,), jnp.float32)
        for i in range(ROWS_W // L):
            sl = pl.ds(i * L, L)
            l0 = lab0_v[sl]
            l1 = lab1_v[sl]
            for c in range(NCLS):
                add = (jnp.where(l0 == c, b0, zero)
                       + jnp.where(l1 == c, b1, zero))
                lg_v[c, sl] = lg_v[c, sl] + add
        pltpu.sync_copy(lg_v, out_hbm.at[:, pl.ds(base, ROWS_W)])

    return _sc_bias


def _row2d(v):
    return v.reshape(1, -1)


def kernel(X, y, W_rf, b_rf, pca_mean, pca_comp, W0, b0, W1, b1,
           W2, b2, W3, b3, nn_bias0, nn_bias1):
    nq = N // RQ
    w3p = jnp.zeros((HID, NPAD), jnp.float32).at[:, :NCLS].set(W3)
    b3p = jnp.zeros((NPAD, 1), jnp.float32).at[:NCLS, 0].set(b3)

    full = lambda shape: pl.BlockSpec(shape, lambda i: (0, 0))
    h1, logits = pl.pallas_call(
        _mlp_body,
        grid=(nq,),
        in_specs=[
            pl.BlockSpec((RQ, D_IN), lambda i: (i, 0)),
            full((D_RF, D_IN)), full((1, D_RF)), full((1, D_RF)),
            full((D_PCA, D_RF)),
            full((HID, HID)), full((1, HID)),
            full((HID, HID)), full((1, HID)),
            full((HID, HID)), full((1, HID)),
            full((HID, NPAD)), full((NPAD, 1)),
        ],
        out_specs=[
            pl.BlockSpec((RQ, HID), lambda i: (i, 0)),
            pl.BlockSpec((NPAD, RQ), lambda i: (0, i)),
        ],
        out_shape=[
            jax.ShapeDtypeStruct((N, HID), jnp.float32),
            jax.ShapeDtypeStruct((NPAD, N), jnp.float32),
        ],
        compiler_params=pltpu.CompilerParams(
            dimension_semantics=("arbitrary",)),
    )(X, W_rf, _row2d(b_rf), _row2d(pca_mean), pca_comp,
      W0, _row2d(b0), W1, _row2d(b1), W2, _row2d(b2), w3p, b3p)

    idx0, idx1 = pl.pallas_call(
        _knn_body,
        grid=(nq,),
        in_specs=[
            pl.BlockSpec((RQ, D_IN), lambda i: (i, 0)),
            pl.BlockSpec((RQ, HID), lambda i: (i, 0)),
            full((N, D_IN)), full((N, HID)),
        ],
        out_specs=[
            pl.BlockSpec((1, 1, RQ), lambda i: (i, 0, 0)),
            pl.BlockSpec((1, 1, RQ), lambda i: (i, 0, 0)),
        ],
        out_shape=[
            jax.ShapeDtypeStruct((nq, 1, RQ), jnp.int32),
            jax.ShapeDtypeStruct((nq, 1, RQ), jnp.int32),
        ],
        compiler_params=pltpu.CompilerParams(
            dimension_semantics=("arbitrary",)),
    )(X, h1, X, h1)

    b0v = jnp.full((L,), nn_bias0, dtype=jnp.float32)
    b1v = jnp.full((L,), nn_bias1, dtype=jnp.float32)
    out16 = _sc_bias_kernel()(logits, idx0.reshape(N), idx1.reshape(N),
                              y, b0v, b1v)
    return out16[:NCLS, :].T
